# baseline (speedup 1.0000x reference)
import functools
import jax
import jax.numpy as jnp
from jax import lax
from jax.experimental import pallas as pl
from jax.experimental.pallas import tpu as pltpu
from jax.experimental.pallas import tpu_sc as plsc

B, N, GX, GY = 2, 100000, 480, 360
FEA, RES = 9, 2
BN_PTS = B * N
VG = B * GX * GY
BLK = 4096
NBLK = (BN_PTS + BLK - 1) // BLK

PB = GX * GY
TR = PB // 16
SUB = 224
NSUB = (TR + SUB - 1) // SUB
LASTR = TR - (NSUB - 1) * SUB
CH = 2048
NPB_PAD = 102400
NCH_A = NPB_PAD // CH
RING = 4096
SL = 2048
GB = 64
FX = 384
FV = FX // 16
NPROC = NBLK * BLK


def _stats_kernel(x_ref, stat_ref):
    i = pl.program_id(0)

    @pl.when(i == 0)
    def _():
        stat_ref[...] = jnp.zeros_like(stat_ref)

    x = x_ref[...]
    rows = jax.lax.broadcasted_iota(jnp.int32, (BLK, 1), 0) + i * BLK
    valid = rows < BN_PTS
    x = jnp.where(valid, x, 0.0)
    pos = x[:, :FEA]
    s = jnp.sum(pos, axis=0, keepdims=True)
    s2 = jnp.sum(pos * pos, axis=0, keepdims=True)
    stat_ref[0:1, :FEA] += s
    stat_ref[1:2, :FEA] += s2


def _layer_kernel(f_out, with_relu_affine, x_ref, w_ref, c_ref, a_ref,
                  t_ref, stat_ref):
    i = pl.program_id(0)

    @pl.when(i == 0)
    def _():
        stat_ref[...] = jnp.zeros_like(stat_ref)

    x = x_ref[...]
    if with_relu_affine:
        x = jnp.maximum(x * a_ref[0:1, :x.shape[1]] + a_ref[1:2, :x.shape[1]], 0.0)
    t = jnp.dot(x, w_ref[...], preferred_element_type=jnp.float32)
    t = t + c_ref[0:1, :t.shape[1]]
    t_ref[...] = t
    rows = jax.lax.broadcasted_iota(jnp.int32, (BLK, 1), 0) + i * BLK
    valid = rows < BN_PTS
    tm = jnp.where(valid, t, 0.0)
    stat_ref[0:1, :f_out] += jnp.sum(tm, axis=0, keepdims=True)
    stat_ref[1:2, :f_out] += jnp.sum(tm * tm, axis=0, keepdims=True)


def _final_kernel(x_ref, w_ref, c_ref, a_ref, pt_ref, t_ref):
    x = x_ref[...]
    x = jnp.maximum(x * a_ref[0:1, :x.shape[1]] + a_ref[1:2, :x.shape[1]], 0.0)
    t = jnp.dot(x, w_ref[...], preferred_element_type=jnp.float32)
    t_ref[:, pl.ds(0, 256)] = t + c_ref[0:1, :t.shape[1]]
    res = pt_ref[:, FEA:FEA + RES]
    t_ref[:, pl.ds(256, 128)] = jnp.concatenate(
        [res, jnp.zeros((BLK, 128 - RES), jnp.float32)], axis=1)


def _run_stats(pt2d):
    return pl.pallas_call(
        _stats_kernel,
        grid=(NBLK,),
        in_specs=[pl.BlockSpec((BLK, FEA + RES), lambda i: (i, 0))],
        out_specs=pl.BlockSpec((8, 128), lambda i: (0, 0)),
        out_shape=jax.ShapeDtypeStruct((8, 128), jnp.float32),
    )(pt2d)


def _run_layer(x, w, c, a, f_out, with_relu_affine, f_in_blk):
    kfn = functools.partial(_layer_kernel, f_out, with_relu_affine)
    t, stat = pl.pallas_call(
        kfn,
        grid=(NBLK,),
        in_specs=[
            pl.BlockSpec((BLK, f_in_blk), lambda i: (i, 0)),
            pl.BlockSpec(w.shape, lambda i: (0, 0)),
            pl.BlockSpec(c.shape, lambda i: (0, 0)),
            pl.BlockSpec(a.shape, lambda i: (0, 0)),
        ],
        out_specs=[
            pl.BlockSpec((BLK, f_out), lambda i: (i, 0)),
            pl.BlockSpec((8, 128), lambda i: (0, 0)),
        ],
        out_shape=[
            jax.ShapeDtypeStruct((NBLK * BLK, f_out), jnp.float32),
            jax.ShapeDtypeStruct((8, 128), jnp.float32),
        ],
    )(x, w, c, a)
    return t, stat


def _run_final(x, w, c, a, pt2d, f_in_blk):
    return pl.pallas_call(
        _final_kernel,
        grid=(NBLK,),
        in_specs=[
            pl.BlockSpec((BLK, f_in_blk), lambda i: (i, 0)),
            pl.BlockSpec(w.shape, lambda i: (0, 0)),
            pl.BlockSpec(c.shape, lambda i: (0, 0)),
            pl.BlockSpec(a.shape, lambda i: (0, 0)),
            pl.BlockSpec((BLK, FEA + RES), lambda i: (i, 0)),
        ],
        out_specs=pl.BlockSpec((BLK, FX), lambda i: (i, 0)),
        out_shape=jax.ShapeDtypeStruct((NPROC, FX), jnp.float32),
    )(x, w, c, a, pt2d)


SENT = 0x40000000

_GDN = lax.GatherDimensionNumbers(
    offset_dims=(), collapsed_slice_dims=(0,), start_index_map=(0,))


def _perm(x, idx):
    return lax.gather(x, idx[:, None], _GDN, slice_sizes=(1,),
                      mode=lax.GatherScatterMode.PROMISE_IN_BOUNDS)


def _pfx(m, iota):
    x = jnp.where(m, 1, 0)
    for k in (1, 2, 4, 8):
        sh = _perm(x, jnp.maximum(iota - k, 0))
        x = x + jnp.where(iota >= k, sh, 0)
    return x


def _compact(m, pos, vi, vv, iota):
    r = jnp.where(m, iota + 1 - pos, 0)
    for b in (1, 2, 4, 8):
        src = jnp.minimum(iota + b, 15)
        ti = _perm(vi, src)
        tv = _perm(vv, src)
        tr = _perm(r, src)
        mv = jnp.where((tr & b) != 0, 1, 0) * jnp.where(iota <= 15 - b, 1, 0)
        vi = jnp.where(mv == 1, ti, vi)
        vv = jnp.where(mv == 1, tv, vv)
        r = jnp.where(mv == 1, tr - b, r)
    return vi, vv


def _append(bidx, bvox, base, ci, cv, iota):
    r0 = base & 15
    asl = base - r0
    rot = (iota - r0) & 15
    ri = _perm(ci, rot)
    rv = _perm(cv, rot)
    w0i = bidx[pl.ds(asl, 16)]
    w0v = bvox[pl.ds(asl, 16)]
    bidx[pl.ds(asl, 16)] = jnp.where(iota < r0, w0i, ri)
    bvox[pl.ds(asl, 16)] = jnp.where(iota < r0, w0v, rv)
    bidx[pl.ds(asl + 16, 16)] = ri
    bvox[pl.ds(asl + 16, 16)] = rv


def _sc_scatter_body(vox_hbm, procx_hbm, pooled_hbm, lidx_hbm, lvox_hbm,
                     ids_in, lb_idx, lb_vox, sl_idx, sl_vox, acc, stage, sem):
    c = lax.axis_index("c")
    s = lax.axis_index("s")
    lo = s * TR
    iota = lax.broadcasted_iota(jnp.int32, (16,), 0)
    ninf = jnp.full((16,), -jnp.inf, jnp.float32)
    zer = jnp.zeros((16,), jnp.int32)
    sent = jnp.full((16,), SENT, jnp.int32)

    def _zr(i, _):
        lb_idx[pl.ds(i * 16, 16)] = zer
        lb_vox[pl.ds(i * 16, 16)] = sent
        return 0
    lax.fori_loop(0, (RING + 128) // 16, _zr, 0)

    def _zs(i, _):
        sl_idx[pl.ds(i * 16, 16)] = zer
        sl_vox[pl.ds(i * 16, 16)] = sent
        return 0
    lax.fori_loop(0, (SL + 128) // 16, _zs, 0)

    pbase = c * N

    def _shift_window(_):
        def mv(k, _):
            wi = lb_idx[pl.ds(CH + k * 16, 16)]
            wv = lb_vox[pl.ds(CH + k * 16, 16)]
            lb_idx[pl.ds(k * 16, 16)] = wi
            lb_vox[pl.ds(k * 16, 16)] = wv
            return 0
        lax.fori_loop(0, (RING + 128 - CH) // 16, mv, 0)

    def chunk_a(g, carry):
        wn, flushed = carry
        pltpu.sync_copy(vox_hbm.at[pl.ds(c * NPB_PAD + g * CH, CH)], ids_in)

        def vec_a(i, wn):
            v = ids_in[pl.ds(i * 16, 16)]
            d = v - lo
            m = plsc.bitcast(d, jnp.uint32) < jnp.uint32(TR)
            pos = _pfx(m, iota)
            vi = jnp.where(m, pbase + g * CH + i * 16 + iota, 0)
            vv = jnp.where(m, d, SENT)
            ci, cv = _compact(m, pos, vi, vv, iota)
            _append(lb_idx, lb_vox, wn - flushed * CH, ci, cv, iota)
            return wn + pos[15]

        wn = lax.fori_loop(0, CH // 16, vec_a, wn)

        for _rep in range(2):
            do_flush = wn - flushed * CH >= CH

            @pl.when(do_flush)
            def _():
                pltpu.sync_copy(lb_idx.at[pl.ds(0, CH)],
                                lidx_hbm.at[c, s, pl.ds(flushed * CH, CH)])
                pltpu.sync_copy(lb_vox.at[pl.ds(0, CH)],
                                lvox_hbm.at[c, s, pl.ds(flushed * CH, CH)])
                _shift_window(0)

            flushed = jnp.where(do_flush, flushed + 1, flushed)

        return wn, flushed

    lt, flushed = lax.fori_loop(0, NCH_A, chunk_a,
                                (jnp.int32(0), jnp.int32(0)))

    @pl.when(lt > flushed * CH)
    def _():
        pltpu.sync_copy(lb_idx.at[pl.ds(0, CH)],
                        lidx_hbm.at[c, s, pl.ds(flushed * CH, CH)])
        pltpu.sync_copy(lb_vox.at[pl.ds(0, CH)],
                        lvox_hbm.at[c, s, pl.ds(flushed * CH, CH)])

    nch_b = (lt + CH - 1) // CH
    rowbase = c * PB + s * TR

    def subrange(sr, _):
        lo_s = sr * SUB

        def ini(r, _):
            for u in range(16):
                acc[pl.ds(r * 256 + u * 16, 16)] = ninf
            return 0
        lax.fori_loop(0, ((SUB + 1) * FV + 15) // 16, ini, 0)

        def chunk_b(g, _):
            pltpu.sync_copy(lidx_hbm.at[c, s, pl.ds(g * CH, CH)],
                            lb_idx.at[pl.ds(0, CH)])
            pltpu.sync_copy(lvox_hbm.at[c, s, pl.ds(g * CH, CH)],
                            lb_vox.at[pl.ds(0, CH)])

            def vec_b(i, sn):
                v = lb_vox[pl.ds(i * 16, 16)]
                d = v - lo_s
                m = plsc.bitcast(d, jnp.uint32) < jnp.uint32(SUB)
                pos = _pfx(m, iota)
                vi = jnp.where(m, lb_idx[pl.ds(i * 16, 16)], 0)
                vv = jnp.where(m, v, SENT)
                ci, cv = _compact(m, pos, vi, vv, iota)
                _append(sl_idx, sl_vox, sn, ci, cv, iota)
                return sn + pos[15]

            sn = lax.fori_loop(0, CH // 16, vec_b, jnp.int32(0))
            nb = (sn + GB - 1) // GB

            def batch(b, _):
                pltpu.async_copy(procx_hbm.at[sl_idx.at[pl.ds(b * GB, GB)]],
                                 stage, sem).wait()
                ng = (jnp.minimum(sn - b * GB, GB) + 15) // 16

                def grp(k, _):
                    gb16 = b * GB + k * 16
                    vvec = sl_vox[pl.ds(gb16, 16)]
                    d = vvec - lo_s
                    okv = plsc.bitcast(d, jnp.uint32) < jnp.uint32(SUB)
                    dd = jnp.where(okv, d, SUB)
                    for q in range(16):
                        vq = dd[q]
                        srow = k * 16 + q
                        for j in range(FV):
                            a = acc[pl.ds(vq * FX + j * 16, 16)]
                            t = stage[srow, pl.ds(j * 16, 16)]
                            acc[pl.ds(vq * FX + j * 16, 16)] = (
                                jnp.maximum(a, t))
                    return 0

                lax.fori_loop(0, ng, grp, 0)
                return 0

            lax.fori_loop(0, nb, batch, 0)
            return 0

        lax.fori_loop(0, nch_b, chunk_b, 0)

        row0 = rowbase + sr * SUB

        @pl.when(sr < NSUB - 1)
        def _():
            pltpu.sync_copy(acc.at[pl.ds(0, SUB * FX)],
                            pooled_hbm.at[pl.ds(row0 * FX, SUB * FX)])

        @pl.when(sr == NSUB - 1)
        def _():
            pltpu.sync_copy(acc.at[pl.ds(0, LASTR * FX)],
                            pooled_hbm.at[pl.ds(row0 * FX, LASTR * FX)])

        return 0

    lax.fori_loop(0, NSUB, subrange, 0)


def _run_sc_scatter(vox_pad, procx):
    mesh = plsc.VectorSubcoreMesh(core_axis_name="c", subcore_axis_name="s")
    f = pl.kernel(
        _sc_scatter_body,
        mesh=mesh,
        out_type=[
            jax.ShapeDtypeStruct((VG * FX,), jnp.float32),
            jax.ShapeDtypeStruct((2, 16, NPB_PAD), jnp.int32),
            jax.ShapeDtypeStruct((2, 16, NPB_PAD), jnp.int32),
        ],
        scratch_types=[
            pltpu.VMEM((CH,), jnp.int32),
            pltpu.VMEM((RING + 128,), jnp.int32),
            pltpu.VMEM((RING + 128,), jnp.int32),
            pltpu.VMEM((SL + 128,), jnp.int32),
            pltpu.VMEM((SL + 128,), jnp.int32),
            pltpu.VMEM((86528,), jnp.float32),
            pltpu.VMEM((GB, FX), jnp.float32),
            pltpu.SemaphoreType.DMA,
        ],
    )
    pooled, _, _ = f(vox_pad, procx)
    return pooled.reshape(VG, FX)


def _bn_affine(stat, f, g, be):
    s = stat[0, :f]
    s2 = stat[1, :f]
    m = s / BN_PTS
    v = s2 / BN_PTS - m * m
    inv = g / jnp.sqrt(v + 1e-5)
    a0 = inv
    a1 = be - m * inv
    a = jnp.zeros((2, f), jnp.float32).at[0].set(a0).at[1].set(a1)
    return a


def kernel(pt_fea, xyz_ind, g0, b0, w1, bl1, g1, be1, w2, bl2, g2, be2,
           w3, bl3, g3, be3, w4, bl4, wc, bc):
    pt2d = pt_fea.reshape(BN_PTS, FEA + RES)

    stat0 = _run_stats(pt2d)
    s = stat0[0, :FEA]
    s2 = stat0[1, :FEA]
    m0 = s / BN_PTS
    v0 = s2 / BN_PTS - m0 * m0
    inv0 = g0 / jnp.sqrt(v0 + 1e-5)
    w1f = jnp.zeros((FEA + RES, 32), jnp.float32).at[:FEA].set(inv0[:, None] * w1)
    c1f = ((b0 - m0 * inv0) @ w1 + bl1)[None, :]
    dummy_a = jnp.zeros((2, FEA + RES), jnp.float32)
    t1, stat1 = _run_layer(pt2d, w1f, c1f, dummy_a, 32, False, FEA + RES)

    a1 = _bn_affine(stat1, 32, g1, be1)
    t2, stat2 = _run_layer(t1, w2, bl2[None, :], a1, 64, True, 32)

    a2 = _bn_affine(stat2, 64, g2, be2)
    t3, stat3 = _run_layer(t2, w3, bl3[None, :], a2, 128, True, 64)

    a3 = _bn_affine(stat3, 128, g3, be3)
    procx = _run_final(t3, w4, bl4[None, :], a3, pt2d, 128)

    xy = xyz_ind.astype(jnp.int32)
    vloc = xy[:, :, 0] * GY + xy[:, :, 1]
    vox_pad = jnp.full((B, NPB_PAD), jnp.int32(0x7FFFFFFF))
    vox_pad = vox_pad.at[:, :N].set(vloc).reshape(-1)

    pooled_ext = _run_sc_scatter(vox_pad, procx)
    neg = jnp.float32(-jnp.inf)
    pooled = pooled_ext[:, :256]
    pooledr = pooled_ext[:, 256:256 + RES]
    occ = pooled[:, 0] > neg
    pm = jnp.where(occ[:, None], pooled, 0.0)
    comp = jax.nn.relu(pm @ wc + bc) * occ[:, None]
    resd = jnp.where(occ[:, None], pooledr, 0.0)

    out = comp.reshape(B, GX, GY, 32).transpose(0, 3, 1, 2)
    out = jax.lax.reduce_window(out, -jnp.inf, jax.lax.max,
                                (1, 1, 3, 3), (1, 1, 1, 1), 'SAME')
    resd = resd.reshape(B, GX, GY, RES).transpose(0, 3, 1, 2)
    return jnp.concatenate([out, resd], axis=1)


# speedup vs baseline: 1.3351x; 1.3351x over previous
import functools
import jax
import jax.numpy as jnp
from jax import lax
from jax.experimental import pallas as pl
from jax.experimental.pallas import tpu as pltpu
from jax.experimental.pallas import tpu_sc as plsc

B, N, GX, GY = 2, 100000, 480, 360
FEA, RES = 9, 2
BN_PTS = B * N
VG = B * GX * GY
BLK = 4096
NBLK = (BN_PTS + BLK - 1) // BLK

PB = GX * GY
TR = PB // 16
SUB = 320
NSUB = (TR + SUB - 1) // SUB
LASTR = TR - (NSUB - 1) * SUB
CH = 2048
NPB_PAD = 102400
NCH_A = NPB_PAD // CH
RING = 4096
SL = 2048
GB = 64
FX = 384
FV = FX // 16
FXA = 272
FVA = FXA // 16
NPROC = NBLK * BLK


def _stats_kernel(x_ref, stat_ref):
    i = pl.program_id(0)

    @pl.when(i == 0)
    def _():
        stat_ref[...] = jnp.zeros_like(stat_ref)

    x = x_ref[...]
    rows = jax.lax.broadcasted_iota(jnp.int32, (BLK, 1), 0) + i * BLK
    valid = rows < BN_PTS
    x = jnp.where(valid, x, 0.0)
    pos = x[:, :FEA]
    s = jnp.sum(pos, axis=0, keepdims=True)
    s2 = jnp.sum(pos * pos, axis=0, keepdims=True)
    stat_ref[0:1, :FEA] += s
    stat_ref[1:2, :FEA] += s2


def _layer_kernel(f_out, with_relu_affine, x_ref, w_ref, c_ref, a_ref,
                  t_ref, stat_ref):
    i = pl.program_id(0)

    @pl.when(i == 0)
    def _():
        stat_ref[...] = jnp.zeros_like(stat_ref)

    x = x_ref[...]
    if with_relu_affine:
        x = jnp.maximum(x * a_ref[0:1, :x.shape[1]] + a_ref[1:2, :x.shape[1]], 0.0)
    t = jnp.dot(x, w_ref[...], preferred_element_type=jnp.float32)
    t = t + c_ref[0:1, :t.shape[1]]
    t_ref[...] = t
    rows = jax.lax.broadcasted_iota(jnp.int32, (BLK, 1), 0) + i * BLK
    valid = rows < BN_PTS
    tm = jnp.where(valid, t, 0.0)
    stat_ref[0:1, :f_out] += jnp.sum(tm, axis=0, keepdims=True)
    stat_ref[1:2, :f_out] += jnp.sum(tm * tm, axis=0, keepdims=True)


def _final_kernel(x_ref, w_ref, c_ref, a_ref, pt_ref, t_ref):
    x = x_ref[...]
    x = jnp.maximum(x * a_ref[0:1, :x.shape[1]] + a_ref[1:2, :x.shape[1]], 0.0)
    t = jnp.dot(x, w_ref[...], preferred_element_type=jnp.float32)
    t_ref[:, pl.ds(0, 256)] = t + c_ref[0:1, :t.shape[1]]
    res = pt_ref[:, FEA:FEA + RES]
    t_ref[:, pl.ds(256, 128)] = jnp.concatenate(
        [res, jnp.zeros((BLK, 128 - RES), jnp.float32)], axis=1)


def _run_stats(pt2d):
    return pl.pallas_call(
        _stats_kernel,
        grid=(NBLK,),
        in_specs=[pl.BlockSpec((BLK, FEA + RES), lambda i: (i, 0))],
        out_specs=pl.BlockSpec((8, 128), lambda i: (0, 0)),
        out_shape=jax.ShapeDtypeStruct((8, 128), jnp.float32),
    )(pt2d)


def _run_layer(x, w, c, a, f_out, with_relu_affine, f_in_blk):
    kfn = functools.partial(_layer_kernel, f_out, with_relu_affine)
    t, stat = pl.pallas_call(
        kfn,
        grid=(NBLK,),
        in_specs=[
            pl.BlockSpec((BLK, f_in_blk), lambda i: (i, 0)),
            pl.BlockSpec(w.shape, lambda i: (0, 0)),
            pl.BlockSpec(c.shape, lambda i: (0, 0)),
            pl.BlockSpec(a.shape, lambda i: (0, 0)),
        ],
        out_specs=[
            pl.BlockSpec((BLK, f_out), lambda i: (i, 0)),
            pl.BlockSpec((8, 128), lambda i: (0, 0)),
        ],
        out_shape=[
            jax.ShapeDtypeStruct((NBLK * BLK, f_out), jnp.float32),
            jax.ShapeDtypeStruct((8, 128), jnp.float32),
        ],
    )(x, w, c, a)
    return t, stat


def _run_final(x, w, c, a, pt2d, f_in_blk):
    return pl.pallas_call(
        _final_kernel,
        grid=(NBLK,),
        in_specs=[
            pl.BlockSpec((BLK, f_in_blk), lambda i: (i, 0)),
            pl.BlockSpec(w.shape, lambda i: (0, 0)),
            pl.BlockSpec(c.shape, lambda i: (0, 0)),
            pl.BlockSpec(a.shape, lambda i: (0, 0)),
            pl.BlockSpec((BLK, FEA + RES), lambda i: (i, 0)),
        ],
        out_specs=pl.BlockSpec((BLK, FX), lambda i: (i, 0)),
        out_shape=jax.ShapeDtypeStruct((NPROC, FX), jnp.float32),
    )(x, w, c, a, pt2d)


SENT = 0x40000000

_GDN = lax.GatherDimensionNumbers(
    offset_dims=(), collapsed_slice_dims=(0,), start_index_map=(0,))


def _perm(x, idx):
    return lax.gather(x, idx[:, None], _GDN, slice_sizes=(1,),
                      mode=lax.GatherScatterMode.PROMISE_IN_BOUNDS)


def _pfx(m, iota):
    x = jnp.where(m, 1, 0)
    for k in (1, 2, 4, 8):
        sh = _perm(x, jnp.maximum(iota - k, 0))
        x = x + jnp.where(iota >= k, sh, 0)
    return x


def _compact(m, pos, vi, vv, iota):
    r = jnp.where(m, iota + 1 - pos, 0)
    for b in (1, 2, 4, 8):
        src = jnp.minimum(iota + b, 15)
        ti = _perm(vi, src)
        tv = _perm(vv, src)
        tr = _perm(r, src)
        mv = jnp.where((tr & b) != 0, 1, 0) * jnp.where(iota <= 15 - b, 1, 0)
        vi = jnp.where(mv == 1, ti, vi)
        vv = jnp.where(mv == 1, tv, vv)
        r = jnp.where(mv == 1, tr - b, r)
    return vi, vv


def _append(bidx, bvox, base, ci, cv, iota):
    r0 = base & 15
    asl = base - r0
    rot = (iota - r0) & 15
    ri = _perm(ci, rot)
    rv = _perm(cv, rot)
    w0i = bidx[pl.ds(asl, 16)]
    w0v = bvox[pl.ds(asl, 16)]
    bidx[pl.ds(asl, 16)] = jnp.where(iota < r0, w0i, ri)
    bvox[pl.ds(asl, 16)] = jnp.where(iota < r0, w0v, rv)
    bidx[pl.ds(asl + 16, 16)] = ri
    bvox[pl.ds(asl + 16, 16)] = rv


def _sc_scatter_body(vox_hbm, procx_hbm, pooled_hbm, lidx_hbm, lvox_hbm,
                     ids_in, lb_idx, lb_vox, sl_idx, sl_vox, acc, stage, sem):
    c = lax.axis_index("c")
    s = lax.axis_index("s")
    lo = s * TR
    iota = lax.broadcasted_iota(jnp.int32, (16,), 0)
    ninf = jnp.full((16,), -jnp.inf, jnp.float32)
    zer = jnp.zeros((16,), jnp.int32)
    sent = jnp.full((16,), SENT, jnp.int32)

    def _zr(i, _):
        lb_idx[pl.ds(i * 16, 16)] = zer
        lb_vox[pl.ds(i * 16, 16)] = sent
        return 0
    lax.fori_loop(0, (RING + 128) // 16, _zr, 0)

    def _zs(i, _):
        sl_idx[pl.ds(i * 16, 16)] = zer
        sl_vox[pl.ds(i * 16, 16)] = sent
        return 0
    lax.fori_loop(0, (SL + 128) // 16, _zs, 0)

    pbase = c * N

    def _shift_window(_):
        def mv(k, _):
            wi = lb_idx[pl.ds(CH + k * 16, 16)]
            wv = lb_vox[pl.ds(CH + k * 16, 16)]
            lb_idx[pl.ds(k * 16, 16)] = wi
            lb_vox[pl.ds(k * 16, 16)] = wv
            return 0
        lax.fori_loop(0, (RING + 128 - CH) // 16, mv, 0)

    def chunk_a(g, carry):
        wn, flushed = carry
        pltpu.sync_copy(vox_hbm.at[pl.ds(c * NPB_PAD + g * CH, CH)], ids_in)

        def vec_a(i, wn):
            v = ids_in[pl.ds(i * 16, 16)]
            d = v - lo
            m = plsc.bitcast(d, jnp.uint32) < jnp.uint32(TR)
            pos = _pfx(m, iota)
            vi = jnp.where(m, pbase + g * CH + i * 16 + iota, 0)
            vv = jnp.where(m, d, SENT)
            ci, cv = _compact(m, pos, vi, vv, iota)
            _append(lb_idx, lb_vox, wn - flushed * CH, ci, cv, iota)
            return wn + pos[15]

        wn = lax.fori_loop(0, CH // 16, vec_a, wn)

        for _rep in range(2):
            do_flush = wn - flushed * CH >= CH

            @pl.when(do_flush)
            def _():
                pltpu.sync_copy(lb_idx.at[pl.ds(0, CH)],
                                lidx_hbm.at[c, s, pl.ds(flushed * CH, CH)])
                pltpu.sync_copy(lb_vox.at[pl.ds(0, CH)],
                                lvox_hbm.at[c, s, pl.ds(flushed * CH, CH)])
                _shift_window(0)

            flushed = jnp.where(do_flush, flushed + 1, flushed)

        return wn, flushed

    lt, flushed = lax.fori_loop(0, NCH_A, chunk_a,
                                (jnp.int32(0), jnp.int32(0)))

    @pl.when(lt > flushed * CH)
    def _():
        pltpu.sync_copy(lb_idx.at[pl.ds(0, CH)],
                        lidx_hbm.at[c, s, pl.ds(flushed * CH, CH)])
        pltpu.sync_copy(lb_vox.at[pl.ds(0, CH)],
                        lvox_hbm.at[c, s, pl.ds(flushed * CH, CH)])

    nch_b = (lt + CH - 1) // CH
    rowbase = c * PB + s * TR

    def subrange(sr, _):
        lo_s = sr * SUB

        def ini(r, _):
            for u in range(16):
                acc[pl.ds(r * 256 + u * 16, 16)] = ninf
            return 0
        lax.fori_loop(0, ((SUB + 1) * FVA + 15) // 16, ini, 0)

        def chunk_b(g, _):
            pltpu.sync_copy(lidx_hbm.at[c, s, pl.ds(g * CH, CH)],
                            lb_idx.at[pl.ds(0, CH)])
            pltpu.sync_copy(lvox_hbm.at[c, s, pl.ds(g * CH, CH)],
                            lb_vox.at[pl.ds(0, CH)])

            def vec_b(i, sn):
                v = lb_vox[pl.ds(i * 16, 16)]
                d = v - lo_s
                m = plsc.bitcast(d, jnp.uint32) < jnp.uint32(SUB)
                pos = _pfx(m, iota)
                vi = jnp.where(m, lb_idx[pl.ds(i * 16, 16)], 0)
                vv = jnp.where(m, v, SENT)
                ci, cv = _compact(m, pos, vi, vv, iota)
                _append(sl_idx, sl_vox, sn, ci, cv, iota)
                return sn + pos[15]

            sn = lax.fori_loop(0, CH // 16, vec_b, jnp.int32(0))
            nb = (sn + GB - 1) // GB

            def batch(b, _):
                pltpu.async_copy(procx_hbm.at[sl_idx.at[pl.ds(b * GB, GB)]],
                                 stage, sem).wait()
                ng = (jnp.minimum(sn - b * GB, GB) + 15) // 16

                def grp(k, _):
                    gb16 = b * GB + k * 16
                    vvec = sl_vox[pl.ds(gb16, 16)]
                    d = vvec - lo_s
                    okv = plsc.bitcast(d, jnp.uint32) < jnp.uint32(SUB)
                    dd = jnp.where(okv, d, SUB)
                    for q in range(16):
                        vq = dd[q]
                        srow = k * 16 + q
                        for j in range(FVA):
                            a = acc[pl.ds(vq * FXA + j * 16, 16)]
                            t = stage[srow, pl.ds(j * 16, 16)]
                            acc[pl.ds(vq * FXA + j * 16, 16)] = (
                                jnp.maximum(a, t))
                    return 0

                lax.fori_loop(0, ng, grp, 0)
                return 0

            lax.fori_loop(0, nb, batch, 0)
            return 0

        lax.fori_loop(0, nch_b, chunk_b, 0)

        row0 = rowbase + sr * SUB

        @pl.when(sr < NSUB - 1)
        def _():
            pltpu.sync_copy(acc.at[pl.ds(0, SUB * FXA)],
                            pooled_hbm.at[pl.ds(row0 * FXA, SUB * FXA)])

        @pl.when(sr == NSUB - 1)
        def _():
            pltpu.sync_copy(acc.at[pl.ds(0, LASTR * FXA)],
                            pooled_hbm.at[pl.ds(row0 * FXA, LASTR * FXA)])

        return 0

    lax.fori_loop(0, NSUB, subrange, 0)


def _run_sc_scatter(vox_pad, procx):
    mesh = plsc.VectorSubcoreMesh(core_axis_name="c", subcore_axis_name="s")
    f = pl.kernel(
        _sc_scatter_body,
        mesh=mesh,
        out_type=[
            jax.ShapeDtypeStruct((VG * FXA,), jnp.float32),
            jax.ShapeDtypeStruct((2, 16, NPB_PAD), jnp.int32),
            jax.ShapeDtypeStruct((2, 16, NPB_PAD), jnp.int32),
        ],
        scratch_types=[
            pltpu.VMEM((CH,), jnp.int32),
            pltpu.VMEM((RING + 128,), jnp.int32),
            pltpu.VMEM((RING + 128,), jnp.int32),
            pltpu.VMEM((SL + 128,), jnp.int32),
            pltpu.VMEM((SL + 128,), jnp.int32),
            pltpu.VMEM((87552,), jnp.float32),
            pltpu.VMEM((GB, FX), jnp.float32),
            pltpu.SemaphoreType.DMA,
        ],
    )
    pooled, _, _ = f(vox_pad, procx)
    return pooled.reshape(VG, FXA)


def _bn_affine(stat, f, g, be):
    s = stat[0, :f]
    s2 = stat[1, :f]
    m = s / BN_PTS
    v = s2 / BN_PTS - m * m
    inv = g / jnp.sqrt(v + 1e-5)
    a0 = inv
    a1 = be - m * inv
    a = jnp.zeros((2, f), jnp.float32).at[0].set(a0).at[1].set(a1)
    return a


def kernel(pt_fea, xyz_ind, g0, b0, w1, bl1, g1, be1, w2, bl2, g2, be2,
           w3, bl3, g3, be3, w4, bl4, wc, bc):
    pt2d = pt_fea.reshape(BN_PTS, FEA + RES)

    stat0 = _run_stats(pt2d)
    s = stat0[0, :FEA]
    s2 = stat0[1, :FEA]
    m0 = s / BN_PTS
    v0 = s2 / BN_PTS - m0 * m0
    inv0 = g0 / jnp.sqrt(v0 + 1e-5)
    w1f = jnp.zeros((FEA + RES, 32), jnp.float32).at[:FEA].set(inv0[:, None] * w1)
    c1f = ((b0 - m0 * inv0) @ w1 + bl1)[None, :]
    dummy_a = jnp.zeros((2, FEA + RES), jnp.float32)
    t1, stat1 = _run_layer(pt2d, w1f, c1f, dummy_a, 32, False, FEA + RES)

    a1 = _bn_affine(stat1, 32, g1, be1)
    t2, stat2 = _run_layer(t1, w2, bl2[None, :], a1, 64, True, 32)

    a2 = _bn_affine(stat2, 64, g2, be2)
    t3, stat3 = _run_layer(t2, w3, bl3[None, :], a2, 128, True, 64)

    a3 = _bn_affine(stat3, 128, g3, be3)
    procx = _run_final(t3, w4, bl4[None, :], a3, pt2d, 128)

    xy = xyz_ind.astype(jnp.int32)
    vloc = xy[:, :, 0] * GY + xy[:, :, 1]
    vox_pad = jnp.full((B, NPB_PAD), jnp.int32(0x7FFFFFFF))
    vox_pad = vox_pad.at[:, :N].set(vloc).reshape(-1)

    pooled_ext = _run_sc_scatter(vox_pad, procx)
    neg = jnp.float32(-jnp.inf)
    pooled = pooled_ext[:, :256]
    pooledr = pooled_ext[:, 256:256 + RES]
    occ = pooled[:, 0] > neg
    pm = jnp.where(occ[:, None], pooled, 0.0)
    comp = jax.nn.relu(pm @ wc + bc) * occ[:, None]
    resd = jnp.where(occ[:, None], pooledr, 0.0)

    out = comp.reshape(B, GX, GY, 32).transpose(0, 3, 1, 2)
    out = jax.lax.reduce_window(out, -jnp.inf, jax.lax.max,
                                (1, 1, 3, 3), (1, 1, 1, 1), 'SAME')
    resd = resd.reshape(B, GX, GY, RES).transpose(0, 3, 1, 2)
    return jnp.concatenate([out, resd], axis=1)


# speedup vs baseline: 1.6068x; 1.2035x over previous
import functools
import jax
import jax.numpy as jnp
from jax import lax
from jax.experimental import pallas as pl
from jax.experimental.pallas import tpu as pltpu
from jax.experimental.pallas import tpu_sc as plsc

B, N, GX, GY = 2, 100000, 480, 360
FEA, RES = 9, 2
BN_PTS = B * N
VG = B * GX * GY
BLK = 4096
NBLK = (BN_PTS + BLK - 1) // BLK

PB = GX * GY
TR = PB // 16
SUB = 368
NSUB = (TR + SUB - 1) // SUB
LASTR = TR - (NSUB - 1) * SUB
CH = 2048
NPB_PAD = 102400
NCH_A = NPB_PAD // CH
RING = 4096
SL = 2048
GB = 32
FX = 384
FV = FX // 16
FXA = 272
FVA = FXA // 16
NPROC = NBLK * BLK


def _stats_kernel(x_ref, stat_ref):
    i = pl.program_id(0)

    @pl.when(i == 0)
    def _():
        stat_ref[...] = jnp.zeros_like(stat_ref)

    x = x_ref[...]
    rows = jax.lax.broadcasted_iota(jnp.int32, (BLK, 1), 0) + i * BLK
    valid = rows < BN_PTS
    x = jnp.where(valid, x, 0.0)
    pos = x[:, :FEA]
    s = jnp.sum(pos, axis=0, keepdims=True)
    s2 = jnp.sum(pos * pos, axis=0, keepdims=True)
    stat_ref[0:1, :FEA] += s
    stat_ref[1:2, :FEA] += s2


def _layer_kernel(f_out, with_relu_affine, x_ref, w_ref, c_ref, a_ref,
                  t_ref, stat_ref):
    i = pl.program_id(0)

    @pl.when(i == 0)
    def _():
        stat_ref[...] = jnp.zeros_like(stat_ref)

    x = x_ref[...]
    if with_relu_affine:
        x = jnp.maximum(x * a_ref[0:1, :x.shape[1]] + a_ref[1:2, :x.shape[1]], 0.0)
    t = jnp.dot(x, w_ref[...], preferred_element_type=jnp.float32)
    t = t + c_ref[0:1, :t.shape[1]]
    t_ref[...] = t
    rows = jax.lax.broadcasted_iota(jnp.int32, (BLK, 1), 0) + i * BLK
    valid = rows < BN_PTS
    tm = jnp.where(valid, t, 0.0)
    stat_ref[0:1, :f_out] += jnp.sum(tm, axis=0, keepdims=True)
    stat_ref[1:2, :f_out] += jnp.sum(tm * tm, axis=0, keepdims=True)


def _final_kernel(x_ref, w_ref, c_ref, a_ref, pt_ref, t_ref):
    x = x_ref[...]
    x = jnp.maximum(x * a_ref[0:1, :x.shape[1]] + a_ref[1:2, :x.shape[1]], 0.0)
    t = jnp.dot(x, w_ref[...], preferred_element_type=jnp.float32)
    t_ref[:, pl.ds(0, 256)] = t + c_ref[0:1, :t.shape[1]]
    res = pt_ref[:, FEA:FEA + RES]
    t_ref[:, pl.ds(256, 128)] = jnp.concatenate(
        [res, jnp.zeros((BLK, 128 - RES), jnp.float32)], axis=1)


def _run_stats(pt2d):
    return pl.pallas_call(
        _stats_kernel,
        grid=(NBLK,),
        in_specs=[pl.BlockSpec((BLK, FEA + RES), lambda i: (i, 0))],
        out_specs=pl.BlockSpec((8, 128), lambda i: (0, 0)),
        out_shape=jax.ShapeDtypeStruct((8, 128), jnp.float32),
    )(pt2d)


def _run_layer(x, w, c, a, f_out, with_relu_affine, f_in_blk):
    kfn = functools.partial(_layer_kernel, f_out, with_relu_affine)
    t, stat = pl.pallas_call(
        kfn,
        grid=(NBLK,),
        in_specs=[
            pl.BlockSpec((BLK, f_in_blk), lambda i: (i, 0)),
            pl.BlockSpec(w.shape, lambda i: (0, 0)),
            pl.BlockSpec(c.shape, lambda i: (0, 0)),
            pl.BlockSpec(a.shape, lambda i: (0, 0)),
        ],
        out_specs=[
            pl.BlockSpec((BLK, f_out), lambda i: (i, 0)),
            pl.BlockSpec((8, 128), lambda i: (0, 0)),
        ],
        out_shape=[
            jax.ShapeDtypeStruct((NBLK * BLK, f_out), jnp.float32),
            jax.ShapeDtypeStruct((8, 128), jnp.float32),
        ],
    )(x, w, c, a)
    return t, stat


def _run_final(x, w, c, a, pt2d, f_in_blk):
    return pl.pallas_call(
        _final_kernel,
        grid=(NBLK,),
        in_specs=[
            pl.BlockSpec((BLK, f_in_blk), lambda i: (i, 0)),
            pl.BlockSpec(w.shape, lambda i: (0, 0)),
            pl.BlockSpec(c.shape, lambda i: (0, 0)),
            pl.BlockSpec(a.shape, lambda i: (0, 0)),
            pl.BlockSpec((BLK, FEA + RES), lambda i: (i, 0)),
        ],
        out_specs=pl.BlockSpec((BLK, FX), lambda i: (i, 0)),
        out_shape=jax.ShapeDtypeStruct((NPROC, FX), jnp.float32),
    )(x, w, c, a, pt2d)


SENT = 0x40000000

_GDN = lax.GatherDimensionNumbers(
    offset_dims=(), collapsed_slice_dims=(0,), start_index_map=(0,))


def _perm(x, idx):
    return lax.gather(x, idx[:, None], _GDN, slice_sizes=(1,),
                      mode=lax.GatherScatterMode.PROMISE_IN_BOUNDS)


def _pfx(m, iota):
    x = jnp.where(m, 1, 0)
    for k in (1, 2, 4, 8):
        sh = _perm(x, jnp.maximum(iota - k, 0))
        x = x + jnp.where(iota >= k, sh, 0)
    return x


def _compact(m, pos, vi, vv, iota):
    r = jnp.where(m, iota + 1 - pos, 0)
    for b in (1, 2, 4, 8):
        src = jnp.minimum(iota + b, 15)
        ti = _perm(vi, src)
        tv = _perm(vv, src)
        tr = _perm(r, src)
        mv = jnp.where((tr & b) != 0, 1, 0) * jnp.where(iota <= 15 - b, 1, 0)
        vi = jnp.where(mv == 1, ti, vi)
        vv = jnp.where(mv == 1, tv, vv)
        r = jnp.where(mv == 1, tr - b, r)
    return vi, vv


def _append(bidx, bvox, base, ci, cv, iota):
    r0 = base & 15
    asl = base - r0
    rot = (iota - r0) & 15
    ri = _perm(ci, rot)
    rv = _perm(cv, rot)
    w0i = bidx[pl.ds(asl, 16)]
    w0v = bvox[pl.ds(asl, 16)]
    bidx[pl.ds(asl, 16)] = jnp.where(iota < r0, w0i, ri)
    bvox[pl.ds(asl, 16)] = jnp.where(iota < r0, w0v, rv)
    bidx[pl.ds(asl + 16, 16)] = ri
    bvox[pl.ds(asl + 16, 16)] = rv


def _sc_scatter_body(vox_hbm, procx_hbm, pooled_hbm, lidx_hbm, lvox_hbm,
                     ids_in, lb_idx, lb_vox, sl_idx, sl_vox, acc, stage, sem):
    c = lax.axis_index("c")
    s = lax.axis_index("s")
    lo = s * TR
    iota = lax.broadcasted_iota(jnp.int32, (16,), 0)
    ninf = jnp.full((16,), -jnp.inf, jnp.float32)
    zer = jnp.zeros((16,), jnp.int32)
    sent = jnp.full((16,), SENT, jnp.int32)

    def _zr(i, _):
        lb_idx[pl.ds(i * 16, 16)] = zer
        lb_vox[pl.ds(i * 16, 16)] = sent
        return 0
    lax.fori_loop(0, (RING + 128) // 16, _zr, 0)

    def _zs(i, _):
        sl_idx[pl.ds(i * 16, 16)] = zer
        sl_vox[pl.ds(i * 16, 16)] = sent
        return 0
    lax.fori_loop(0, (SL + 128) // 16, _zs, 0)

    pbase = c * N

    def _shift_window(_):
        def mv(k, _):
            wi = lb_idx[pl.ds(CH + k * 16, 16)]
            wv = lb_vox[pl.ds(CH + k * 16, 16)]
            lb_idx[pl.ds(k * 16, 16)] = wi
            lb_vox[pl.ds(k * 16, 16)] = wv
            return 0
        lax.fori_loop(0, (RING + 128 - CH) // 16, mv, 0)

    def chunk_a(g, carry):
        wn, flushed = carry
        pltpu.sync_copy(vox_hbm.at[pl.ds(c * NPB_PAD + g * CH, CH)], ids_in)

        def vec_a(i, wn):
            v = ids_in[pl.ds(i * 16, 16)]
            d = v - lo
            m = plsc.bitcast(d, jnp.uint32) < jnp.uint32(TR)
            pos = _pfx(m, iota)
            vi = jnp.where(m, pbase + g * CH + i * 16 + iota, 0)
            vv = jnp.where(m, d, SENT)
            ci, cv = _compact(m, pos, vi, vv, iota)
            _append(lb_idx, lb_vox, wn - flushed * CH, ci, cv, iota)
            return wn + pos[15]

        wn = lax.fori_loop(0, CH // 16, vec_a, wn)

        for _rep in range(2):
            do_flush = wn - flushed * CH >= CH

            @pl.when(do_flush)
            def _():
                pltpu.sync_copy(lb_idx.at[pl.ds(0, CH)],
                                lidx_hbm.at[c, s, pl.ds(flushed * CH, CH)])
                pltpu.sync_copy(lb_vox.at[pl.ds(0, CH)],
                                lvox_hbm.at[c, s, pl.ds(flushed * CH, CH)])
                _shift_window(0)

            flushed = jnp.where(do_flush, flushed + 1, flushed)

        return wn, flushed

    lt, flushed = lax.fori_loop(0, NCH_A, chunk_a,
                                (jnp.int32(0), jnp.int32(0)))

    @pl.when(lt > flushed * CH)
    def _():
        pltpu.sync_copy(lb_idx.at[pl.ds(0, CH)],
                        lidx_hbm.at[c, s, pl.ds(flushed * CH, CH)])
        pltpu.sync_copy(lb_vox.at[pl.ds(0, CH)],
                        lvox_hbm.at[c, s, pl.ds(flushed * CH, CH)])

    nch_b = (lt + CH - 1) // CH
    rowbase = c * PB + s * TR

    def subrange(sr, _):
        lo_s = sr * SUB

        def ini(r, _):
            for u in range(16):
                acc[pl.ds(r * 256 + u * 16, 16)] = ninf
            return 0
        lax.fori_loop(0, ((SUB + 1) * FVA + 15) // 16, ini, 0)

        def chunk_b(g, _):
            pltpu.sync_copy(lidx_hbm.at[c, s, pl.ds(g * CH, CH)],
                            lb_idx.at[pl.ds(0, CH)])
            pltpu.sync_copy(lvox_hbm.at[c, s, pl.ds(g * CH, CH)],
                            lb_vox.at[pl.ds(0, CH)])

            def vec_b(i, sn):
                v = lb_vox[pl.ds(i * 16, 16)]
                d = v - lo_s
                m = plsc.bitcast(d, jnp.uint32) < jnp.uint32(SUB)
                pos = _pfx(m, iota)
                vi = jnp.where(m, lb_idx[pl.ds(i * 16, 16)], 0)
                vv = jnp.where(m, v, SENT)
                ci, cv = _compact(m, pos, vi, vv, iota)
                _append(sl_idx, sl_vox, sn, ci, cv, iota)
                return sn + pos[15]

            sn = lax.fori_loop(0, CH // 16, vec_b, jnp.int32(0))
            nb = (sn + GB - 1) // GB

            def batch(b, _):
                pltpu.async_copy(procx_hbm.at[sl_idx.at[pl.ds(b * GB, GB)]],
                                 stage, sem).wait()
                ng = (jnp.minimum(sn - b * GB, GB) + 15) // 16

                def grp(k, _):
                    gb16 = b * GB + k * 16
                    vvec = sl_vox[pl.ds(gb16, 16)]
                    d = vvec - lo_s
                    okv = plsc.bitcast(d, jnp.uint32) < jnp.uint32(SUB)
                    dd = jnp.where(okv, d, SUB)
                    for q in range(16):
                        vq = dd[q]
                        srow = k * 16 + q
                        for j in range(FVA):
                            a = acc[pl.ds(vq * FXA + j * 16, 16)]
                            t = stage[srow, pl.ds(j * 16, 16)]
                            acc[pl.ds(vq * FXA + j * 16, 16)] = (
                                jnp.maximum(a, t))
                    return 0

                lax.fori_loop(0, ng, grp, 0)
                return 0

            lax.fori_loop(0, nb, batch, 0)
            return 0

        lax.fori_loop(0, nch_b, chunk_b, 0)

        row0 = rowbase + sr * SUB

        @pl.when(sr < NSUB - 1)
        def _():
            pltpu.sync_copy(acc.at[pl.ds(0, SUB * FXA)],
                            pooled_hbm.at[pl.ds(row0 * FXA, SUB * FXA)])

        @pl.when(sr == NSUB - 1)
        def _():
            pltpu.sync_copy(acc.at[pl.ds(0, LASTR * FXA)],
                            pooled_hbm.at[pl.ds(row0 * FXA, LASTR * FXA)])

        return 0

    lax.fori_loop(0, NSUB, subrange, 0)


def _run_sc_scatter(vox_pad, procx):
    mesh = plsc.VectorSubcoreMesh(core_axis_name="c", subcore_axis_name="s")
    f = pl.kernel(
        _sc_scatter_body,
        mesh=mesh,
        out_type=[
            jax.ShapeDtypeStruct((VG * FXA,), jnp.float32),
            jax.ShapeDtypeStruct((2, 16, NPB_PAD), jnp.int32),
            jax.ShapeDtypeStruct((2, 16, NPB_PAD), jnp.int32),
        ],
        scratch_types=[
            pltpu.VMEM((CH,), jnp.int32),
            pltpu.VMEM((RING + 128,), jnp.int32),
            pltpu.VMEM((RING + 128,), jnp.int32),
            pltpu.VMEM((SL + 128,), jnp.int32),
            pltpu.VMEM((SL + 128,), jnp.int32),
            pltpu.VMEM((100608,), jnp.float32),
            pltpu.VMEM((GB, FX), jnp.float32),
            pltpu.SemaphoreType.DMA,
        ],
    )
    pooled, _, _ = f(vox_pad, procx)
    return pooled.reshape(VG, FXA)


def _bn_affine(stat, f, g, be):
    s = stat[0, :f]
    s2 = stat[1, :f]
    m = s / BN_PTS
    v = s2 / BN_PTS - m * m
    inv = g / jnp.sqrt(v + 1e-5)
    a0 = inv
    a1 = be - m * inv
    a = jnp.zeros((2, f), jnp.float32).at[0].set(a0).at[1].set(a1)
    return a


def kernel(pt_fea, xyz_ind, g0, b0, w1, bl1, g1, be1, w2, bl2, g2, be2,
           w3, bl3, g3, be3, w4, bl4, wc, bc):
    pt2d = pt_fea.reshape(BN_PTS, FEA + RES)

    stat0 = _run_stats(pt2d)
    s = stat0[0, :FEA]
    s2 = stat0[1, :FEA]
    m0 = s / BN_PTS
    v0 = s2 / BN_PTS - m0 * m0
    inv0 = g0 / jnp.sqrt(v0 + 1e-5)
    w1f = jnp.zeros((FEA + RES, 32), jnp.float32).at[:FEA].set(inv0[:, None] * w1)
    c1f = ((b0 - m0 * inv0) @ w1 + bl1)[None, :]
    dummy_a = jnp.zeros((2, FEA + RES), jnp.float32)
    t1, stat1 = _run_layer(pt2d, w1f, c1f, dummy_a, 32, False, FEA + RES)

    a1 = _bn_affine(stat1, 32, g1, be1)
    t2, stat2 = _run_layer(t1, w2, bl2[None, :], a1, 64, True, 32)

    a2 = _bn_affine(stat2, 64, g2, be2)
    t3, stat3 = _run_layer(t2, w3, bl3[None, :], a2, 128, True, 64)

    a3 = _bn_affine(stat3, 128, g3, be3)
    procx = _run_final(t3, w4, bl4[None, :], a3, pt2d, 128)

    xy = xyz_ind.astype(jnp.int32)
    vloc = xy[:, :, 0] * GY + xy[:, :, 1]
    vox_pad = jnp.full((B, NPB_PAD), jnp.int32(0x7FFFFFFF))
    vox_pad = vox_pad.at[:, :N].set(vloc).reshape(-1)

    pooled_ext = _run_sc_scatter(vox_pad, procx)
    neg = jnp.float32(-jnp.inf)
    pooled = pooled_ext[:, :256]
    pooledr = pooled_ext[:, 256:256 + RES]
    occ = pooled[:, 0] > neg
    pm = jnp.where(occ[:, None], pooled, 0.0)
    comp = jax.nn.relu(pm @ wc + bc) * occ[:, None]
    resd = jnp.where(occ[:, None], pooledr, 0.0)

    out = comp.reshape(B, GX, GY, 32).transpose(0, 3, 1, 2)
    out = jax.lax.reduce_window(out, -jnp.inf, jax.lax.max,
                                (1, 1, 3, 3), (1, 1, 1, 1), 'SAME')
    resd = resd.reshape(B, GX, GY, RES).transpose(0, 3, 1, 2)
    return jnp.concatenate([out, resd], axis=1)
